# baseline (speedup 1.0000x reference)
import functools

import jax
import jax.numpy as jnp
from jax import lax
from jax.experimental import pallas as pl
from jax.experimental.pallas import tpu as pltpu
from jax.experimental.pallas import tpu_sc as plsc

NV = 10000
NKP = 100000
NE = 160000

VP = 10240
SLEN = 10496
KPP = 100352
EP = 163840
D = 304
DN = 16
DG = 384
DB = 128
DE = 320

NC, NS, L = 2, 16, 16
NW = NC * NS

F32 = jnp.float32


def _inorm_relu(x, w):
    W = x.shape[-1]
    if w == W:
        m = jnp.mean(x, -1, keepdims=True)
        d = x - m
        v = jnp.mean(d * d, -1, keepdims=True)
        return jnp.maximum(d * lax.rsqrt(v + 1e-5), 0.0)
    mask = lax.broadcasted_iota(jnp.int32, x.shape, 1) < w
    xm = jnp.where(mask, x, 0.0)
    m = jnp.sum(xm, -1, keepdims=True) * (1.0 / w)
    d = jnp.where(mask, x - m, 0.0)
    v = jnp.sum(d * d, -1, keepdims=True) * (1.0 / w)
    y = d * lax.rsqrt(v + 1e-5)
    return jnp.where(mask, jnp.maximum(y, 0.0), 0.0)


def _padw(wb, ri, ro):
    Wm, b = wb
    fi, fo = Wm.shape
    Wp = jnp.zeros((ri, ro), F32).at[:fi, :fo].set(Wm)
    bp = jnp.zeros((ro,), F32).at[:fo].set(b)
    return Wp, bp


def _full_spec(shape):
    return pl.BlockSpec(shape, lambda i: (0,) * len(shape))


def _row_spec(blk, width):
    return pl.BlockSpec((blk, width), lambda i: (i, 0))


def _run_mlp(x, wbs, widths, blk):
    n = x.shape[0]

    def body(*refs):
        x_ref, wrefs, o_ref = refs[0], refs[1:-1], refs[-1]
        xv = x_ref[...]
        for k, w in enumerate(widths):
            Wr = wrefs[2 * k][...]
            xv = _inorm_relu(
                jnp.dot(xv.astype(Wr.dtype), Wr, preferred_element_type=F32)
                + wrefs[2 * k + 1][...][None, :], w)
        o_ref[...] = xv

    args = [x]
    in_specs = [_row_spec(blk, x.shape[1])]
    for (Wp, bp) in wbs:
        args += [Wp, bp]
        in_specs += [_full_spec(Wp.shape), _full_spec(bp.shape)]
    out_w = wbs[-1][0].shape[1]
    return pl.pallas_call(
        body,
        grid=(n // blk,),
        in_specs=in_specs,
        out_specs=_row_spec(blk, out_w),
        out_shape=jax.ShapeDtypeStruct((n, out_w), F32),
    )(*args)


def _vertex_kernel(s, posb, wh1, bh1, wh2, bh2, wf1s, bf1, wf1x):
    VB = 1024

    def body(s_ref, p_ref, wh1r, bh1r, wh2r, bh2r, wf1r, bf1r, wf1xr,
             ta_ref, tb_ref):
        sv = s_ref[...]
        pv = p_ref[...]
        t = _inorm_relu(jnp.dot(sv, wh1r[...], preferred_element_type=F32)
                        + bh1r[...][None, :], 64)
        delta = _inorm_relu(
            jnp.dot(t, wh2r[...], preferred_element_type=F32)
            + bh2r[...][None, :], 3)
        su = (jnp.dot(sv, wf1r[...], preferred_element_type=F32)
              + bf1r[...][None, :]
              + jnp.dot(pv, wf1xr[...], preferred_element_type=F32))
        ta_ref[...] = jnp.concatenate(
            [su, jnp.zeros((VB, DG - D), F32)], axis=-1)
        tb_ref[...] = jnp.concatenate(
            [pv + delta, jnp.zeros((VB, DB - DN), F32)], axis=-1)

    return pl.pallas_call(
        body,
        grid=(VP // VB,),
        in_specs=[_row_spec(VB, D), _row_spec(VB, DN),
                  _full_spec(wh1.shape), _full_spec(bh1.shape),
                  _full_spec(wh2.shape), _full_spec(bh2.shape),
                  _full_spec(wf1s.shape), _full_spec(bf1.shape),
                  _full_spec(wf1x.shape)],
        out_specs=(_row_spec(VB, DG), _row_spec(VB, DB)),
        out_shape=(jax.ShapeDtypeStruct((VP, DG), F32),
                   jax.ShapeDtypeStruct((VP, DB), F32)),
    )(s, posb, wh1, bh1, wh2, bh2, wf1s, bf1, wf1x)


def _edge_kernel(TAg, TBg, wf1xb, wf2, bf2):
    EB = 2048
    NR = TAg.shape[0]

    def body(ta_ref, tb_ref, wf1xr, wf2r, bf2r, o_ref):
        pre1 = (ta_ref[...].astype(F32)
                - jnp.dot(tb_ref[...], wf1xr[...],
                          preferred_element_type=F32))
        u = _inorm_relu(pre1, 300)
        o_ref[...] = _inorm_relu(
            jnp.dot(u, wf2r[...], preferred_element_type=F32)
            + bf2r[...][None, :], 300)

    return pl.pallas_call(
        body,
        grid=(NR // EB,),
        in_specs=[_row_spec(EB, DG), _row_spec(EB, DB),
                  _full_spec(wf1xb.shape), _full_spec(wf2.shape),
                  _full_spec(bf2.shape)],
        out_specs=_row_spec(EB, D),
        out_shape=jax.ShapeDtypeStruct((NR, D), F32),
    )(TAg, TBg, wf1xb, wf2, bf2)


def _g_kernel(agg, s, w1, b1, w2, b2, residual):
    VB = 1024

    def body(a_ref, s_ref, w1r, b1r, w2r, b2r, o_ref):
        u = _inorm_relu(jnp.dot(a_ref[...], w1r[...],
                                preferred_element_type=F32)
                        + b1r[...][None, :], 300)
        y = _inorm_relu(jnp.dot(u, w2r[...], preferred_element_type=F32)
                        + b2r[...][None, :], 300)
        if residual:
            y = y + s_ref[...]
        o_ref[...] = y

    return pl.pallas_call(
        body,
        grid=(VP // VB,),
        in_specs=[_row_spec(VB, agg.shape[1]), _row_spec(VB, D),
                  _full_spec(w1.shape), _full_spec(b1.shape),
                  _full_spec(w2.shape), _full_spec(b2.shape)],
        out_specs=_row_spec(VB, D),
        out_shape=jax.ShapeDtypeStruct((VP, D), F32),
    )(agg, s, w1, b1, w2, b2)


def _aggr2_kernel(agg0, agg1, w1, b1, w2, b2):
    VB = 1024

    def body(a0_ref, a1_ref, w1r, b1r, w2r, b2r, o_ref):
        a = jnp.maximum(a0_ref[...], a1_ref[...])
        u = _inorm_relu(jnp.dot(a, w1r[...], preferred_element_type=F32)
                        + b1r[...][None, :], 300)
        o_ref[...] = _inorm_relu(
            jnp.dot(u, w2r[...], preferred_element_type=F32)
            + b2r[...][None, :], 300)

    return pl.pallas_call(
        body,
        grid=(VP // VB,),
        in_specs=[_row_spec(VB, D), _row_spec(VB, D),
                  _full_spec(w1.shape), _full_spec(b1.shape),
                  _full_spec(w2.shape), _full_spec(b2.shape)],
        out_specs=_row_spec(VB, D),
        out_shape=jax.ShapeDtypeStruct((VP, D), F32),
    )(agg0, agg1, w1, b1, w2, b2)


def _head_kernel(s, cls_wbs, loc_wbs):
    VB = 1024
    flat = list(cls_wbs)
    for lw in loc_wbs:
        flat += list(lw)

    def body(*refs):
        s_ref = refs[0]
        wr = refs[1:-2]
        cls_ref, reg_ref = refs[-2], refs[-1]
        sv = s_ref[...]
        c = _inorm_relu(jnp.dot(sv, wr[0][...], preferred_element_type=F32)
                        + wr[1][...][None, :], 64)
        cls_ref[...] = _inorm_relu(
            jnp.dot(c, wr[2][...], preferred_element_type=F32)
            + wr[3][...][None, :], 4)
        outs = []
        for i in range(4):
            base = 4 + 6 * i
            x = _inorm_relu(
                jnp.dot(sv, wr[base][...], preferred_element_type=F32)
                + wr[base + 1][...][None, :], 64)
            x = _inorm_relu(
                jnp.dot(x, wr[base + 2][...], preferred_element_type=F32)
                + wr[base + 3][...][None, :], 64)
            x = _inorm_relu(
                jnp.dot(x, wr[base + 4][...], preferred_element_type=F32)
                + wr[base + 5][...][None, :], 7)
            outs.append(x)
        reg_ref[...] = jnp.concatenate(outs, axis=-1)

    in_specs = [_row_spec(VB, D)]
    args = [s]
    for a in flat:
        args.append(a)
        in_specs.append(_full_spec(a.shape))
    return pl.pallas_call(
        body,
        grid=(VP // VB,),
        in_specs=in_specs,
        out_specs=(_row_spec(VB, 8), _row_spec(VB, 32)),
        out_shape=(jax.ShapeDtypeStruct((VP, 8), F32),
                   jax.ShapeDtypeStruct((VP, 32), F32)),
    )(*args)


def _sc_mesh():
    return plsc.VectorSubcoreMesh(core_axis_name="c", subcore_axis_name="s",
                                  num_cores=NC, num_subcores=NS)


def _sc_gather2(tabA, idxA, tabB, idxB):
    WA = tabA.shape[1]
    WB = tabB.shape[1]
    DTA = tabA.dtype
    DTB = tabB.dtype
    NR = idxA.shape[0]
    CH = 64
    NBUF = 3
    RPT = NR // NW
    NCHK = RPT // CH

    @functools.partial(
        pl.kernel,
        out_type=(jax.ShapeDtypeStruct((NR, WA), DTA),
                  jax.ShapeDtypeStruct((NR, WB), DTB)),
        mesh=_sc_mesh(),
        scratch_types=(
            [pltpu.VMEM((CH,), jnp.int32)] * (2 * NBUF)
            + [pltpu.VMEM((CH, WA), DTA)] * NBUF
            + [pltpu.VMEM((CH, WB), DTB)] * NBUF
            + [pltpu.SemaphoreType.DMA] * (6 * NBUF)
        ),
    )
    def body(tabA_hbm, idxA_hbm, tabB_hbm, idxB_hbm, outA_hbm, outB_hbm,
             *scr):
        wid = lax.axis_index("s") * NC + lax.axis_index("c")
        base = wid * RPT
        idxv = (scr[0:NBUF], scr[NBUF:2 * NBUF])
        bufv = (scr[2 * NBUF:3 * NBUF], scr[3 * NBUF:4 * NBUF])
        sems = scr[4 * NBUF:]
        tabs = (tabA_hbm, tabB_hbm)
        idxh = (idxA_hbm, idxB_hbm)
        outh = (outA_hbm, outB_hbm)
        sis = (sems[0:NBUF], sems[NBUF:2 * NBUF])
        sgs = (sems[2 * NBUF:3 * NBUF], sems[3 * NBUF:4 * NBUF])
        sws = (sems[4 * NBUF:5 * NBUF], sems[5 * NBUF:6 * NBUF])

        def idx_start(t, k):
            return pltpu.async_copy(idxh[t].at[pl.ds(base + k * CH, CH)],
                                    idxv[t][k % NBUF], sis[t][k % NBUF])

        def gather_start(t, k):
            return pltpu.async_copy(tabs[t].at[idxv[t][k % NBUF]],
                                    bufv[t][k % NBUF], sgs[t][k % NBUF])

        def wb_start(t, k):
            return pltpu.async_copy(bufv[t][k % NBUF],
                                    outh[t].at[pl.ds(base + k * CH, CH)],
                                    sws[t][k % NBUF])

        idx_d = [[None] * NCHK, [None] * NCHK]
        g_d = [[None] * NCHK, [None] * NCHK]
        w_d = [[None] * NCHK, [None] * NCHK]
        for t in (0, 1):
            for j in range(min(NBUF, NCHK)):
                idx_d[t][j] = idx_start(t, j)
        started = [0, 0]

        def ensure_started(t, upto):
            while started[t] <= min(upto, NCHK - 1):
                j = started[t]
                idx_d[t][j].wait()
                if j - NBUF >= 0:
                    w_d[t][j - NBUF].wait()
                g_d[t][j] = gather_start(t, j)
                started[t] += 1

        for k in range(NCHK):
            for t in (0, 1):
                ensure_started(t, k + NBUF - 1)
            for t in (0, 1):
                g_d[t][k].wait()
                w_d[t][k] = wb_start(t, k)
                if k + NBUF < NCHK:
                    idx_d[t][k + NBUF] = idx_start(t, k + NBUF)
        for t in (0, 1):
            for j in range(max(0, NCHK - NBUF), NCHK):
                w_d[t][j].wait()

    return body(tabA, idxA, tabB, idxB)


def _sc_segmax(data, starts):
    CH = 256
    VPW = VP // NW
    NP = data.shape[0]
    Wd = data.shape[1]
    DT = data.dtype
    lanes = 32 if DT == jnp.bfloat16 else 16
    NACC = Wd // lanes

    @functools.partial(
        pl.kernel,
        out_type=jax.ShapeDtypeStruct((VP, Wd), DT),
        mesh=_sc_mesh(),
        scratch_types=[
            pltpu.VMEM((352,), jnp.int32),
            pltpu.VMEM((CH, Wd), DT),
            pltpu.VMEM((64, Wd), DT),
        ],
    )
    def body(data_hbm, starts_hbm, out_hbm, st_v, buf_v, vout_v):
        wid = lax.axis_index("s") * NC + lax.axis_index("c")
        v0 = wid * VPW
        pltpu.sync_copy(starts_hbm.at[pl.ds(v0, 352)], st_v)
        r0 = st_v[pl.ds(0, 16)][0]
        pltpu.sync_copy(
            data_hbm.at[pl.ds(
                pl.multiple_of(jnp.minimum((r0 // CH) * CH, NP - CH), CH),
                CH)],
            buf_v)

        def vbody(v, _):
            sv = st_v[pl.ds(v, 16)]
            s0 = sv[0]
            cnt = sv[1] - s0
            acc0 = tuple(jnp.zeros((lanes,), DT) for _ in range(NACC))

            def rbody(i, acc):
                rc = s0 + i
                o = lax.rem(rc, CH)

                @pl.when(o == 0)
                def _():
                    pltpu.sync_copy(
                        data_hbm.at[pl.ds(pl.multiple_of(rc, CH), CH)],
                        buf_v)

                return tuple(
                    jnp.maximum(acc[c], buf_v[o, pl.ds(c * lanes, lanes)])
                    for c in range(NACC))

            acc = lax.fori_loop(0, cnt, rbody, acc0)
            vm = lax.rem(v, 64)
            for c in range(NACC):
                vout_v[vm, pl.ds(c * lanes, lanes)] = acc[c]

            @pl.when(vm == 63)
            def _():
                pltpu.sync_copy(
                    vout_v,
                    out_hbm.at[pl.ds(pl.multiple_of(v0 + v - 63, 64), 64)])

            return 0

        lax.fori_loop(0, VPW, vbody, 0, unroll=False)

    return body(data, starts)


def kernel(key_points, pos, params, key_points_lookup, edge_index):
    src = edge_index[0]
    dst = edge_index[1]
    perm = jnp.argsort(src)
    src_s = src[perm]
    dst_s = dst[perm]
    src_sp = jnp.zeros((EP,), jnp.int32).at[:NE].set(src_s)
    dst_sp = jnp.zeros((EP,), jnp.int32).at[:NE].set(dst_s)
    estarts = jnp.searchsorted(src_s, jnp.arange(NV + 1, dtype=jnp.int32),
                               side="left").astype(jnp.int32)
    estarts_p = jnp.full((SLEN,), NE, jnp.int32).at[:NV + 1].set(estarts)
    kstarts_p = (jnp.full((SLEN,), NKP, jnp.int32)
                 .at[:NV].set(key_points_lookup.astype(jnp.int32)))

    kp_pad = jnp.zeros((KPP, 8), F32).at[:NKP, :4].set(key_points)
    pos_pad = jnp.zeros((VP, DN), F32).at[:NV, :3].set(pos)

    init_ch = [8, 32, 64, 128, D]
    init_wbs = [_padw(params["init"][i], init_ch[i], init_ch[i + 1])
                for i in range(4)]
    aggr_wbs = [_padw(params["aggr"][0], D, D), _padw(params["aggr"][1], D, D)]
    cls_wbs = _padw(params["cls"][0], D, 64) + _padw(params["cls"][1], 64, 8)
    loc_wbs = [
        _padw(loc[0], D, 64) + _padw(loc[1], 64, 64) + _padw(loc[2], 64, 8)
        for loc in params["loc"]
    ]

    layers = []
    for lp in params["layers"]:
        wh1, bh1 = _padw(lp["h"][0], D, 64)
        wh2, bh2 = _padw(lp["h"][1], 64, DN)
        Wf1, bf1 = lp["f"][0]
        wf1x = jnp.zeros((DN, D), F32).at[:3, :300].set(Wf1[:3])
        wf1xb = jnp.zeros((DB, DG), F32).at[:3, :300].set(Wf1[:3])
        wf1s = jnp.zeros((D, D), F32).at[:300, :300].set(Wf1[3:])
        bf1p = jnp.zeros((D,), F32).at[:300].set(bf1)
        wf2 = jnp.zeros((DG, D), F32).at[:300, :300].set(lp["f"][1][0])
        bf2 = jnp.zeros((D,), F32).at[:300].set(lp["f"][1][1])
        wg1, bg1 = _padw(lp["g"][0], D, D)
        wg2, bg2 = _padw(lp["g"][1], D, D)
        layers.append((wh1, bh1, wh2, bh2, wf1x, wf1xb, wf1s, bf1p, wf2, bf2,
                       wg1, bg1, wg2, bg2))

    KH = 49152
    kst0 = jnp.minimum(kstarts_p, KH)
    kst1 = jnp.clip(kstarts_p, KH, NKP) - KH
    kpf0 = _run_mlp(kp_pad[:KH], init_wbs, [32, 64, 128, 300], blk=2048)
    aggk0 = _sc_segmax(kpf0, kst0)
    kpf1 = _run_mlp(kp_pad[KH:], init_wbs, [32, 64, 128, 300], blk=2048)
    aggk1 = _sc_segmax(kpf1, kst1)
    s = _aggr2_kernel(aggk0, aggk1, aggr_wbs[0][0], aggr_wbs[0][1],
                      aggr_wbs[1][0], aggr_wbs[1][1])

    for (wh1, bh1, wh2, bh2, wf1x, wf1xb, wf1s, bf1p, wf2, bf2,
         wg1, bg1, wg2, bg2) in layers:
        TA, TB = _vertex_kernel(s, pos_pad, wh1, bh1, wh2, bh2, wf1s, bf1p,
                                wf1x)
        TAg, TBg = _sc_gather2(TA, dst_sp, TB, src_sp)
        e = _edge_kernel(TAg, TBg, wf1xb, wf2, bf2)
        agg = _sc_segmax(e, estarts_p)
        s = _g_kernel(agg, s, wg1, bg1, wg2, bg2, residual=True)

    cls_p, reg_p = _head_kernel(s, cls_wbs, loc_wbs)
    cls_pred = cls_p[:NV, :4]
    reg_pred = jnp.concatenate([reg_p[:NV, 8 * i:8 * i + 7] for i in range(4)],
                               axis=-1)
    return (cls_pred, reg_pred)


# speedup vs baseline: 1.0199x; 1.0199x over previous
import functools

import jax
import jax.numpy as jnp
from jax import lax
from jax.experimental import pallas as pl
from jax.experimental.pallas import tpu as pltpu
from jax.experimental.pallas import tpu_sc as plsc

NV = 10000
NKP = 100000
NE = 160000

VP = 10240
SLEN = 10496
KPP = 100352
EP = 163840
D = 304
DN = 16
DG = 384
DB = 128
DE = 320

NC, NS, L = 2, 16, 16
NW = NC * NS

F32 = jnp.float32


def _inorm_relu(x, w):
    W = x.shape[-1]
    if w == W:
        m = jnp.mean(x, -1, keepdims=True)
        d = x - m
        v = jnp.mean(d * d, -1, keepdims=True)
        return jnp.maximum(d * lax.rsqrt(v + 1e-5), 0.0)
    mask = lax.broadcasted_iota(jnp.int32, x.shape, 1) < w
    xm = jnp.where(mask, x, 0.0)
    m = jnp.sum(xm, -1, keepdims=True) * (1.0 / w)
    d = jnp.where(mask, x - m, 0.0)
    v = jnp.sum(d * d, -1, keepdims=True) * (1.0 / w)
    y = d * lax.rsqrt(v + 1e-5)
    return jnp.where(mask, jnp.maximum(y, 0.0), 0.0)


def _padw(wb, ri, ro):
    Wm, b = wb
    fi, fo = Wm.shape
    Wp = jnp.zeros((ri, ro), F32).at[:fi, :fo].set(Wm)
    bp = jnp.zeros((ro,), F32).at[:fo].set(b)
    return Wp, bp


def _full_spec(shape):
    return pl.BlockSpec(shape, lambda i: (0,) * len(shape))


def _row_spec(blk, width):
    return pl.BlockSpec((blk, width), lambda i: (i, 0))


def _run_mlp(x, wbs, widths, blk):
    n = x.shape[0]

    def body(*refs):
        x_ref, wrefs, o_ref = refs[0], refs[1:-1], refs[-1]
        xv = x_ref[...]
        for k, w in enumerate(widths):
            Wr = wrefs[2 * k][...]
            xv = _inorm_relu(
                jnp.dot(xv.astype(Wr.dtype), Wr, preferred_element_type=F32)
                + wrefs[2 * k + 1][...][None, :], w)
        o_ref[...] = xv

    args = [x]
    in_specs = [_row_spec(blk, x.shape[1])]
    for (Wp, bp) in wbs:
        args += [Wp, bp]
        in_specs += [_full_spec(Wp.shape), _full_spec(bp.shape)]
    out_w = wbs[-1][0].shape[1]
    return pl.pallas_call(
        body,
        grid=(n // blk,),
        in_specs=in_specs,
        out_specs=_row_spec(blk, out_w),
        out_shape=jax.ShapeDtypeStruct((n, out_w), F32),
    )(*args)


def _vertex_kernel(s, posb, wh1, bh1, wh2, bh2, wf1s, bf1, wf1x):
    VB = 1024

    def body(s_ref, p_ref, wh1r, bh1r, wh2r, bh2r, wf1r, bf1r, wf1xr,
             ta_ref, tb_ref):
        sv = s_ref[...]
        pv = p_ref[...]
        t = _inorm_relu(jnp.dot(sv, wh1r[...], preferred_element_type=F32)
                        + bh1r[...][None, :], 64)
        delta = _inorm_relu(
            jnp.dot(t, wh2r[...], preferred_element_type=F32)
            + bh2r[...][None, :], 3)
        su = (jnp.dot(sv, wf1r[...], preferred_element_type=F32)
              + bf1r[...][None, :]
              + jnp.dot(pv, wf1xr[...], preferred_element_type=F32))
        ta_ref[...] = jnp.concatenate(
            [su, jnp.zeros((VB, DG - D), F32)], axis=-1)
        tb_ref[...] = jnp.concatenate(
            [pv + delta, jnp.zeros((VB, DB - DN), F32)], axis=-1)

    return pl.pallas_call(
        body,
        grid=(VP // VB,),
        in_specs=[_row_spec(VB, D), _row_spec(VB, DN),
                  _full_spec(wh1.shape), _full_spec(bh1.shape),
                  _full_spec(wh2.shape), _full_spec(bh2.shape),
                  _full_spec(wf1s.shape), _full_spec(bf1.shape),
                  _full_spec(wf1x.shape)],
        out_specs=(_row_spec(VB, DG), _row_spec(VB, DB)),
        out_shape=(jax.ShapeDtypeStruct((VP, DG), F32),
                   jax.ShapeDtypeStruct((VP, DB), F32)),
    )(s, posb, wh1, bh1, wh2, bh2, wf1s, bf1, wf1x)


def _edge_kernel(TAg, TBg, wf1xb, wf2, bf2):
    EB = 2048
    NR = TAg.shape[0]

    def body(ta_ref, tb_ref, wf1xr, wf2r, bf2r, o_ref):
        pre1 = (ta_ref[...].astype(F32)
                - jnp.dot(tb_ref[...], wf1xr[...],
                          preferred_element_type=F32))
        u = _inorm_relu(pre1, 300)
        o_ref[...] = _inorm_relu(
            jnp.dot(u, wf2r[...], preferred_element_type=F32)
            + bf2r[...][None, :], 300)

    return pl.pallas_call(
        body,
        grid=(NR // EB,),
        in_specs=[_row_spec(EB, DG), _row_spec(EB, DB),
                  _full_spec(wf1xb.shape), _full_spec(wf2.shape),
                  _full_spec(bf2.shape)],
        out_specs=_row_spec(EB, D),
        out_shape=jax.ShapeDtypeStruct((NR, D), F32),
    )(TAg, TBg, wf1xb, wf2, bf2)


def _g_kernel(agg, s, w1, b1, w2, b2, residual):
    VB = 1024

    def body(a_ref, s_ref, w1r, b1r, w2r, b2r, o_ref):
        u = _inorm_relu(jnp.dot(a_ref[...], w1r[...],
                                preferred_element_type=F32)
                        + b1r[...][None, :], 300)
        y = _inorm_relu(jnp.dot(u, w2r[...], preferred_element_type=F32)
                        + b2r[...][None, :], 300)
        if residual:
            y = y + s_ref[...]
        o_ref[...] = y

    return pl.pallas_call(
        body,
        grid=(VP // VB,),
        in_specs=[_row_spec(VB, agg.shape[1]), _row_spec(VB, D),
                  _full_spec(w1.shape), _full_spec(b1.shape),
                  _full_spec(w2.shape), _full_spec(b2.shape)],
        out_specs=_row_spec(VB, D),
        out_shape=jax.ShapeDtypeStruct((VP, D), F32),
    )(agg, s, w1, b1, w2, b2)


def _head_kernel(s, cls_wbs, loc_wbs):
    VB = 1024
    flat = list(cls_wbs)
    for lw in loc_wbs:
        flat += list(lw)

    def body(*refs):
        s_ref = refs[0]
        wr = refs[1:-2]
        cls_ref, reg_ref = refs[-2], refs[-1]
        sv = s_ref[...]
        c = _inorm_relu(jnp.dot(sv, wr[0][...], preferred_element_type=F32)
                        + wr[1][...][None, :], 64)
        cls_ref[...] = _inorm_relu(
            jnp.dot(c, wr[2][...], preferred_element_type=F32)
            + wr[3][...][None, :], 4)
        outs = []
        for i in range(4):
            base = 4 + 6 * i
            x = _inorm_relu(
                jnp.dot(sv, wr[base][...], preferred_element_type=F32)
                + wr[base + 1][...][None, :], 64)
            x = _inorm_relu(
                jnp.dot(x, wr[base + 2][...], preferred_element_type=F32)
                + wr[base + 3][...][None, :], 64)
            x = _inorm_relu(
                jnp.dot(x, wr[base + 4][...], preferred_element_type=F32)
                + wr[base + 5][...][None, :], 7)
            outs.append(x)
        reg_ref[...] = jnp.concatenate(outs, axis=-1)

    in_specs = [_row_spec(VB, D)]
    args = [s]
    for a in flat:
        args.append(a)
        in_specs.append(_full_spec(a.shape))
    return pl.pallas_call(
        body,
        grid=(VP // VB,),
        in_specs=in_specs,
        out_specs=(_row_spec(VB, 8), _row_spec(VB, 32)),
        out_shape=(jax.ShapeDtypeStruct((VP, 8), F32),
                   jax.ShapeDtypeStruct((VP, 32), F32)),
    )(*args)


def _sc_mesh():
    return plsc.VectorSubcoreMesh(core_axis_name="c", subcore_axis_name="s",
                                  num_cores=NC, num_subcores=NS)


def _sc_gather2(tabA, idxA, tabB, idxB):
    WA = tabA.shape[1]
    WB = tabB.shape[1]
    DTA = tabA.dtype
    DTB = tabB.dtype
    NR = idxA.shape[0]
    CH = 64
    NBUF = 3
    RPT = NR // NW
    NCHK = RPT // CH

    @functools.partial(
        pl.kernel,
        out_type=(jax.ShapeDtypeStruct((NR, WA), DTA),
                  jax.ShapeDtypeStruct((NR, WB), DTB)),
        mesh=_sc_mesh(),
        scratch_types=(
            [pltpu.VMEM((CH,), jnp.int32)] * (2 * NBUF)
            + [pltpu.VMEM((CH, WA), DTA)] * NBUF
            + [pltpu.VMEM((CH, WB), DTB)] * NBUF
            + [pltpu.SemaphoreType.DMA] * (6 * NBUF)
        ),
    )
    def body(tabA_hbm, idxA_hbm, tabB_hbm, idxB_hbm, outA_hbm, outB_hbm,
             *scr):
        wid = lax.axis_index("s") * NC + lax.axis_index("c")
        base = wid * RPT
        idxv = (scr[0:NBUF], scr[NBUF:2 * NBUF])
        bufv = (scr[2 * NBUF:3 * NBUF], scr[3 * NBUF:4 * NBUF])
        sems = scr[4 * NBUF:]
        tabs = (tabA_hbm, tabB_hbm)
        idxh = (idxA_hbm, idxB_hbm)
        outh = (outA_hbm, outB_hbm)
        sis = (sems[0:NBUF], sems[NBUF:2 * NBUF])
        sgs = (sems[2 * NBUF:3 * NBUF], sems[3 * NBUF:4 * NBUF])
        sws = (sems[4 * NBUF:5 * NBUF], sems[5 * NBUF:6 * NBUF])

        def idx_start(t, k):
            return pltpu.async_copy(idxh[t].at[pl.ds(base + k * CH, CH)],
                                    idxv[t][k % NBUF], sis[t][k % NBUF])

        def gather_start(t, k):
            return pltpu.async_copy(tabs[t].at[idxv[t][k % NBUF]],
                                    bufv[t][k % NBUF], sgs[t][k % NBUF])

        def wb_start(t, k):
            return pltpu.async_copy(bufv[t][k % NBUF],
                                    outh[t].at[pl.ds(base + k * CH, CH)],
                                    sws[t][k % NBUF])

        idx_d = [[None] * NCHK, [None] * NCHK]
        g_d = [[None] * NCHK, [None] * NCHK]
        w_d = [[None] * NCHK, [None] * NCHK]
        for t in (0, 1):
            for j in range(min(NBUF, NCHK)):
                idx_d[t][j] = idx_start(t, j)
        started = [0, 0]

        def ensure_started(t, upto):
            while started[t] <= min(upto, NCHK - 1):
                j = started[t]
                idx_d[t][j].wait()
                if j - NBUF >= 0:
                    w_d[t][j - NBUF].wait()
                g_d[t][j] = gather_start(t, j)
                started[t] += 1

        for k in range(NCHK):
            for t in (0, 1):
                ensure_started(t, k + NBUF - 1)
            for t in (0, 1):
                g_d[t][k].wait()
                w_d[t][k] = wb_start(t, k)
                if k + NBUF < NCHK:
                    idx_d[t][k + NBUF] = idx_start(t, k + NBUF)
        for t in (0, 1):
            for j in range(max(0, NCHK - NBUF), NCHK):
                w_d[t][j].wait()

    return body(tabA, idxA, tabB, idxB)


def _sc_segmax(data, starts):
    CH = 256
    VPW = VP // NW
    NP = data.shape[0]
    Wd = data.shape[1]
    DT = data.dtype
    lanes = 32 if DT == jnp.bfloat16 else 16
    NACC = Wd // lanes

    @functools.partial(
        pl.kernel,
        out_type=jax.ShapeDtypeStruct((VP, Wd), DT),
        mesh=_sc_mesh(),
        scratch_types=[
            pltpu.VMEM((352,), jnp.int32),
            pltpu.VMEM((CH, Wd), DT),
            pltpu.VMEM((64, Wd), DT),
        ],
    )
    def body(data_hbm, starts_hbm, out_hbm, st_v, buf_v, vout_v):
        wid = lax.axis_index("s") * NC + lax.axis_index("c")
        v0 = wid * VPW
        pltpu.sync_copy(starts_hbm.at[pl.ds(v0, 352)], st_v)
        r0 = st_v[pl.ds(0, 16)][0]
        pltpu.sync_copy(
            data_hbm.at[pl.ds(
                pl.multiple_of(jnp.minimum((r0 // CH) * CH, NP - CH), CH),
                CH)],
            buf_v)

        def vbody(v, _):
            sv = st_v[pl.ds(v, 16)]
            s0 = sv[0]
            cnt = sv[1] - s0
            acc0 = tuple(jnp.zeros((lanes,), DT) for _ in range(NACC))

            def rbody(i, acc):
                rc = s0 + i
                o = lax.rem(rc, CH)

                @pl.when(o == 0)
                def _():
                    pltpu.sync_copy(
                        data_hbm.at[pl.ds(pl.multiple_of(rc, CH), CH)],
                        buf_v)

                return tuple(
                    jnp.maximum(acc[c], buf_v[o, pl.ds(c * lanes, lanes)])
                    for c in range(NACC))

            acc = lax.fori_loop(0, cnt, rbody, acc0)
            vm = lax.rem(v, 64)
            for c in range(NACC):
                vout_v[vm, pl.ds(c * lanes, lanes)] = acc[c]

            @pl.when(vm == 63)
            def _():
                pltpu.sync_copy(
                    vout_v,
                    out_hbm.at[pl.ds(pl.multiple_of(v0 + v - 63, 64), 64)])

            return 0

        lax.fori_loop(0, VPW, vbody, 0, unroll=False)

    return body(data, starts)


def kernel(key_points, pos, params, key_points_lookup, edge_index):
    src = edge_index[0]
    dst = edge_index[1]
    perm = jnp.argsort(src)
    src_s = src[perm]
    dst_s = dst[perm]
    src_sp = jnp.zeros((EP,), jnp.int32).at[:NE].set(src_s)
    dst_sp = jnp.zeros((EP,), jnp.int32).at[:NE].set(dst_s)
    estarts = jnp.searchsorted(src_s, jnp.arange(NV + 1, dtype=jnp.int32),
                               side="left").astype(jnp.int32)
    estarts_p = jnp.full((SLEN,), NE, jnp.int32).at[:NV + 1].set(estarts)
    kstarts_p = (jnp.full((SLEN,), NKP, jnp.int32)
                 .at[:NV].set(key_points_lookup.astype(jnp.int32)))

    kp_pad = jnp.zeros((KPP, 8), F32).at[:NKP, :4].set(key_points)
    pos_pad = jnp.zeros((VP, DN), F32).at[:NV, :3].set(pos)

    init_ch = [8, 32, 64, 128, D]
    init_wbs = [_padw(params["init"][i], init_ch[i], init_ch[i + 1])
                for i in range(4)]
    aggr_wbs = [_padw(params["aggr"][0], D, D), _padw(params["aggr"][1], D, D)]
    cls_wbs = _padw(params["cls"][0], D, 64) + _padw(params["cls"][1], 64, 8)
    loc_wbs = [
        _padw(loc[0], D, 64) + _padw(loc[1], 64, 64) + _padw(loc[2], 64, 8)
        for loc in params["loc"]
    ]

    layers = []
    for lp in params["layers"]:
        wh1, bh1 = _padw(lp["h"][0], D, 64)
        wh2, bh2 = _padw(lp["h"][1], 64, DN)
        Wf1, bf1 = lp["f"][0]
        wf1x = jnp.zeros((DN, D), F32).at[:3, :300].set(Wf1[:3])
        wf1xb = jnp.zeros((DB, DG), F32).at[:3, :300].set(Wf1[:3])
        wf1s = jnp.zeros((D, D), F32).at[:300, :300].set(Wf1[3:])
        bf1p = jnp.zeros((D,), F32).at[:300].set(bf1)
        wf2 = jnp.zeros((DG, D), F32).at[:300, :300].set(lp["f"][1][0])
        bf2 = jnp.zeros((D,), F32).at[:300].set(lp["f"][1][1])
        wg1, bg1 = _padw(lp["g"][0], D, D)
        wg2, bg2 = _padw(lp["g"][1], D, D)
        layers.append((wh1, bh1, wh2, bh2, wf1x, wf1xb, wf1s, bf1p, wf2, bf2,
                       wg1, bg1, wg2, bg2))

    kp_feats = _run_mlp(kp_pad, init_wbs, [32, 64, 128, 300], blk=2048)
    agg_kp = _sc_segmax(kp_feats, kstarts_p)
    s = _g_kernel(agg_kp, agg_kp, aggr_wbs[0][0], aggr_wbs[0][1],
                  aggr_wbs[1][0], aggr_wbs[1][1], residual=False)

    for (wh1, bh1, wh2, bh2, wf1x, wf1xb, wf1s, bf1p, wf2, bf2,
         wg1, bg1, wg2, bg2) in layers:
        TA, TB = _vertex_kernel(s, pos_pad, wh1, bh1, wh2, bh2, wf1s, bf1p,
                                wf1x)
        TAg, TBg = _sc_gather2(TA, dst_sp, TB, src_sp)
        e = _edge_kernel(TAg, TBg, wf1xb, wf2, bf2)
        agg = _sc_segmax(e, estarts_p)
        s = _g_kernel(agg, s, wg1, bg1, wg2, bg2, residual=True)

    cls_p, reg_p = _head_kernel(s, cls_wbs, loc_wbs)
    cls_pred = cls_p[:NV, :4]
    reg_pred = jnp.concatenate([reg_p[:NV, 8 * i:8 * i + 7] for i in range(4)],
                               axis=-1)
    return (cls_pred, reg_pred)
